# baseline (speedup 1.0000x reference)
import functools

import jax
import jax.numpy as jnp
from jax import lax
from jax.experimental import pallas as pl
from jax.experimental.pallas import tpu as pltpu
from jax.experimental.pallas import tpu_sc as plsc

_BM = 128
_NW = 32
_TOPK = 2


def _sc_dispatch(hs, inv2, s_max):
    t, h = hs.shape
    tok_w = t // _NW
    ch = 16
    n_it = tok_w // ch
    info = plsc.get_sparse_core_info()
    nc = info.num_cores
    mesh = plsc.VectorSubcoreMesh(core_axis_name="c", subcore_axis_name="s")

    @functools.partial(
        pl.kernel,
        mesh=mesh,
        out_type=jax.ShapeDtypeStruct((s_max, h), jnp.float32),
        scratch_types=[
            pltpu.VMEM((_TOPK, tok_w), jnp.int32),
            pltpu.VMEM((2, ch, h), jnp.float32),
            pltpu.SemaphoreType.DMA,
            pltpu.SemaphoreType.DMA,
            pltpu.SemaphoreType.DMA,
            pltpu.SemaphoreType.DMA,
        ],
    )
    def k(hs_hbm, inv_hbm, out_hbm, idx_v, rows_v, si0, si1, so0, so1):
        wid = lax.axis_index("s") * nc + lax.axis_index("c")
        base = pl.multiple_of(wid * tok_w, 8)
        pltpu.sync_copy(inv_hbm.at[pl.ds(base, tok_w)], idx_v.at[0])
        pltpu.sync_copy(inv_hbm.at[pl.ds(t + base, tok_w)], idx_v.at[1])
        sems_in = (si0, si1)
        sems_out = (so0, so1)

        def in_copy(i, b):
            lo = pl.multiple_of(i * ch, 8)
            return pltpu.make_async_copy(
                hs_hbm.at[pl.ds(base + lo, ch)], rows_v.at[b], sems_in[b])

        def out_copies(i, b):
            lo = pl.multiple_of(i * ch, 8)
            return (
                pltpu.make_async_copy(
                    rows_v.at[b], out_hbm.at[idx_v.at[0, pl.ds(lo, ch)]],
                    sems_out[b]),
                pltpu.make_async_copy(
                    rows_v.at[b], out_hbm.at[idx_v.at[1, pl.ds(lo, ch)]],
                    sems_out[b]),
            )

        in_copy(0, 0).start()
        in_copy(1, 1).start()

        def step(j, carry):
            for b in (0, 1):
                i = j * 2 + b
                in_copy(i, b).wait()
                c0, c1 = out_copies(i, b)
                c0.start()
                c1.start()

                @pl.when(i + 2 < n_it)
                def _():
                    c0.wait()
                    c1.wait()
                    in_copy(i + 2, b).start()

            return carry

        lax.fori_loop(0, n_it // 2, step, 0)
        for b in (0, 1):
            c0, c1 = out_copies(n_it - 2 + b, b)
            c0.wait()
            c1.wait()

    return k(hs, inv2)


def _sc_combine(y, inv2, pk2, res):
    t, h = res.shape
    tok_w = t // _NW
    ch = 8
    n_it = tok_w // ch
    info = plsc.get_sparse_core_info()
    nc = info.num_cores
    mesh = plsc.VectorSubcoreMesh(core_axis_name="c", subcore_axis_name="s")

    @functools.partial(
        pl.kernel,
        mesh=mesh,
        out_type=jax.ShapeDtypeStruct((t, h), jnp.float32),
        scratch_types=[
            pltpu.VMEM((_TOPK, tok_w), jnp.int32),
            pltpu.VMEM((_TOPK, tok_w), jnp.float32),
            pltpu.VMEM((2, ch, h), jnp.float32),
            pltpu.VMEM((2, ch, h), jnp.float32),
            pltpu.VMEM((2, ch, h), jnp.float32),
            pltpu.SemaphoreType.DMA,
            pltpu.SemaphoreType.DMA,
            pltpu.SemaphoreType.DMA,
            pltpu.SemaphoreType.DMA,
        ],
    )
    def k(y_hbm, inv_hbm, pk_hbm, res_hbm, out_hbm,
          idx_v, pk_v, y0_v, y1_v, r_v, si0, si1, sw0, sw1):
        wid = lax.axis_index("s") * nc + lax.axis_index("c")
        base = pl.multiple_of(wid * tok_w, 8)
        pltpu.sync_copy(inv_hbm.at[pl.ds(base, tok_w)], idx_v.at[0])
        pltpu.sync_copy(inv_hbm.at[pl.ds(t + base, tok_w)], idx_v.at[1])
        pltpu.sync_copy(pk_hbm.at[pl.ds(base, tok_w)], pk_v.at[0])
        pltpu.sync_copy(pk_hbm.at[pl.ds(t + base, tok_w)], pk_v.at[1])
        sems_in = (si0, si1)
        sems_w = (sw0, sw1)

        def in_copies(i, b):
            lo = pl.multiple_of(i * ch, 8)
            return (
                pltpu.make_async_copy(
                    y_hbm.at[idx_v.at[0, pl.ds(lo, ch)]], y0_v.at[b],
                    sems_in[b]),
                pltpu.make_async_copy(
                    y_hbm.at[idx_v.at[1, pl.ds(lo, ch)]], y1_v.at[b],
                    sems_in[b]),
                pltpu.make_async_copy(
                    res_hbm.at[pl.ds(base + lo, ch)], r_v.at[b], sems_in[b]),
            )

        def wb_copy(i, b):
            lo = pl.multiple_of(i * ch, 8)
            return pltpu.make_async_copy(
                r_v.at[b], out_hbm.at[pl.ds(base + lo, ch)], sems_w[b])

        for c in in_copies(0, 0):
            c.start()
        for c in in_copies(1, 1):
            c.start()

        dnums = lax.GatherDimensionNumbers(
            offset_dims=(), collapsed_slice_dims=(0,), start_index_map=(0,))

        def step(j, carry):
            for b in (0, 1):
                i = j * 2 + b
                for c in in_copies(i, b):
                    c.wait()
                w0 = pl.multiple_of((i // 2) * 16, 8)
                pk0w = pk_v[0, pl.ds(w0, 16)]
                pk1w = pk_v[1, pl.ds(w0, 16)]
                half = (i % 2) * ch

                def row_loop(r, c2):
                    lane = (jnp.zeros((16,), jnp.int32) + half + r)[:, None]
                    s0 = lax.gather(
                        pk0w, lane, dnums, (1,),
                        mode=lax.GatherScatterMode.PROMISE_IN_BOUNDS)
                    s1 = lax.gather(
                        pk1w, lane, dnums, (1,),
                        mode=lax.GatherScatterMode.PROMISE_IN_BOUNDS)

                    @plsc.parallel_loop(0, h, step=128)
                    def col_loop(c0):
                        for u in range(8):
                            slc = pl.ds(c0 + u * 16, 16)
                            r_v[b, r, slc] = (r_v[b, r, slc]
                                              + y0_v[b, r, slc] * s0
                                              + y1_v[b, r, slc] * s1)

                    return c2

                lax.fori_loop(0, ch, row_loop, 0)
                wb = wb_copy(i, b)
                wb.start()

                @pl.when(i + 2 < n_it)
                def _():
                    c0, c1, c2 = in_copies(i + 2, b)
                    c0.start()
                    c1.start()
                    wb.wait()
                    c2.start()

            return carry

        lax.fori_loop(0, n_it // 2, step, 0)
        wb_copy(n_it - 2, 0).wait()
        wb_copy(n_it - 1, 1).wait()

    return k(y, inv2, pk2, res)


def _cast_body(w_ref, o_ref):
    o_ref[...] = w_ref[...].astype(jnp.bfloat16)


def _cast_w(w):
    e, a, b = w.shape
    return pl.pallas_call(
        _cast_body,
        grid=(e,),
        in_specs=[pl.BlockSpec((1, a, b), lambda i: (i, 0, 0))],
        out_specs=pl.BlockSpec((1, a, b), lambda i: (i, 0, 0)),
        out_shape=jax.ShapeDtypeStruct((e, a, b), jnp.bfloat16),
        compiler_params=pltpu.CompilerParams(
            dimension_semantics=("arbitrary",)),
    )(w)


def _mlp_body(eid_ref, x_ref, w1_ref, w2_ref, y_ref):
    x = x_ref[...].astype(jnp.bfloat16)
    a = jnp.dot(x, w1_ref[0], preferred_element_type=jnp.float32)
    a = jax.nn.gelu(a)
    y_ref[...] = jnp.dot(a.astype(jnp.bfloat16), w2_ref[0],
                         preferred_element_type=jnp.float32)


def _grouped_mlp(xpad, eid, w1b, w2b):
    s_max, h = xpad.shape
    e, _, ff = w1b.shape
    nb = s_max // _BM
    grid_spec = pltpu.PrefetchScalarGridSpec(
        num_scalar_prefetch=1,
        grid=(nb,),
        in_specs=[
            pl.BlockSpec((_BM, h), lambda i, eid_ref: (i, 0)),
            pl.BlockSpec((1, h, ff), lambda i, eid_ref: (eid_ref[i], 0, 0)),
            pl.BlockSpec((1, ff, h), lambda i, eid_ref: (eid_ref[i], 0, 0)),
        ],
        out_specs=pl.BlockSpec((_BM, h), lambda i, eid_ref: (i, 0)),
    )
    return pl.pallas_call(
        _mlp_body,
        grid_spec=grid_spec,
        out_shape=jax.ShapeDtypeStruct((s_max, h), jnp.float32),
        compiler_params=pltpu.CompilerParams(
            dimension_semantics=("arbitrary",)),
    )(eid, xpad, w1b, w2b)


def kernel(hidden_states, mlp_residual, probs, routing_map, W1, W2):
    t, h = hidden_states.shape
    e = W1.shape[0]
    i32 = jnp.int32
    s = t * _TOPK
    nb = s // _BM + e
    s_max = nb * _BM

    rm = routing_map
    rmi = rm.astype(i32)
    counts = jnp.sum(rmi, axis=0)
    g = 128
    rmi3 = rmi.reshape(t // g, g, e).astype(jnp.float32)
    tril_incl = jnp.tril(jnp.ones((g, g), jnp.float32))
    within = jnp.einsum("jk,bke->bje", tril_incl, rmi3,
                        preferred_element_type=jnp.float32)
    bsum = rmi3.sum(axis=1)
    tril_excl = jnp.tril(jnp.ones((t // g, t // g), jnp.float32), k=-1)
    boff = jnp.einsum("jk,ke->je", tril_excl, bsum,
                      preferred_element_type=jnp.float32)
    rank = (within + boff[:, None, :]).astype(i32).reshape(t, e) - rmi
    padded = ((counts + _BM - 1) // _BM) * _BM
    ends = jnp.cumsum(padded)
    off = ends - padded
    pos = (off[None, :] + rank).astype(i32)

    inv0 = jnp.min(jnp.where(rm, pos, s_max + 1), axis=1)
    inv1 = jnp.max(jnp.where(rm, pos, -1), axis=1)
    inv2 = jnp.concatenate([inv0, inv1])

    eidx = lax.broadcasted_iota(i32, (t, e), 1)
    e0 = jnp.min(jnp.where(rm, eidx, e), axis=1)
    e1 = jnp.max(jnp.where(rm, eidx, -1), axis=1)
    pk0 = jnp.sum(jnp.where(eidx == e0[:, None], probs, 0.0), axis=1)
    pk1 = jnp.sum(jnp.where(eidx == e1[:, None], probs, 0.0), axis=1)
    pk2 = jnp.concatenate([pk0, pk1])

    blk_start = jnp.arange(nb, dtype=i32) * _BM
    eid = jnp.minimum(jnp.searchsorted(ends, blk_start, side="right"),
                      e - 1).astype(i32)

    xpad = _sc_dispatch(hidden_states, inv2, s_max)

    w1b = _cast_w(W1)
    w2b = _cast_w(W2)
    y = _grouped_mlp(xpad, eid, w1b, w2b)

    return _sc_combine(y, inv2, pk2, mlp_residual)


# speedup vs baseline: 1.1047x; 1.1047x over previous
import functools

import jax
import jax.numpy as jnp
from jax import lax
from jax.experimental import pallas as pl
from jax.experimental.pallas import tpu as pltpu
from jax.experimental.pallas import tpu_sc as plsc

_BM = 256
_NW = 32
_TOPK = 2


def _sc_dispatch(hs, inv2, s_max):
    t, h = hs.shape
    tok_w = t // _NW
    ch = 16
    n_it = tok_w // ch
    info = plsc.get_sparse_core_info()
    nc = info.num_cores
    mesh = plsc.VectorSubcoreMesh(core_axis_name="c", subcore_axis_name="s")

    @functools.partial(
        pl.kernel,
        mesh=mesh,
        out_type=jax.ShapeDtypeStruct((s_max, h), jnp.float32),
        scratch_types=[
            pltpu.VMEM((_TOPK, tok_w), jnp.int32),
            pltpu.VMEM((2, ch, h), jnp.float32),
            pltpu.SemaphoreType.DMA,
            pltpu.SemaphoreType.DMA,
            pltpu.SemaphoreType.DMA,
            pltpu.SemaphoreType.DMA,
        ],
    )
    def k(hs_hbm, inv_hbm, out_hbm, idx_v, rows_v, si0, si1, so0, so1):
        wid = lax.axis_index("s") * nc + lax.axis_index("c")
        base = pl.multiple_of(wid * tok_w, 8)
        pltpu.sync_copy(inv_hbm.at[pl.ds(base, tok_w)], idx_v.at[0])
        pltpu.sync_copy(inv_hbm.at[pl.ds(t + base, tok_w)], idx_v.at[1])
        sems_in = (si0, si1)
        sems_out = (so0, so1)

        def in_copy(i, b):
            lo = pl.multiple_of(i * ch, 8)
            return pltpu.make_async_copy(
                hs_hbm.at[pl.ds(base + lo, ch)], rows_v.at[b], sems_in[b])

        def out_copies(i, b):
            lo = pl.multiple_of(i * ch, 8)
            return (
                pltpu.make_async_copy(
                    rows_v.at[b], out_hbm.at[idx_v.at[0, pl.ds(lo, ch)]],
                    sems_out[b]),
                pltpu.make_async_copy(
                    rows_v.at[b], out_hbm.at[idx_v.at[1, pl.ds(lo, ch)]],
                    sems_out[b]),
            )

        in_copy(0, 0).start()
        in_copy(1, 1).start()

        def step(j, carry):
            for b in (0, 1):
                i = j * 2 + b
                in_copy(i, b).wait()
                c0, c1 = out_copies(i, b)
                c0.start()
                c1.start()

                @pl.when(i + 2 < n_it)
                def _():
                    c0.wait()
                    c1.wait()
                    in_copy(i + 2, b).start()

            return carry

        lax.fori_loop(0, n_it // 2, step, 0)
        for b in (0, 1):
            c0, c1 = out_copies(n_it - 2 + b, b)
            c0.wait()
            c1.wait()

    return k(hs, inv2)


def _sc_combine(y, inv2, pk2, res):
    t, h = res.shape
    tok_w = t // _NW
    ch = 8
    n_it = tok_w // ch
    info = plsc.get_sparse_core_info()
    nc = info.num_cores
    mesh = plsc.VectorSubcoreMesh(core_axis_name="c", subcore_axis_name="s")

    @functools.partial(
        pl.kernel,
        mesh=mesh,
        out_type=jax.ShapeDtypeStruct((t, h), jnp.float32),
        scratch_types=[
            pltpu.VMEM((_TOPK, tok_w), jnp.int32),
            pltpu.VMEM((_TOPK, tok_w), jnp.float32),
            pltpu.VMEM((2, ch, h), jnp.float32),
            pltpu.VMEM((2, ch, h), jnp.float32),
            pltpu.VMEM((2, ch, h), jnp.float32),
            pltpu.SemaphoreType.DMA,
            pltpu.SemaphoreType.DMA,
            pltpu.SemaphoreType.DMA,
            pltpu.SemaphoreType.DMA,
        ],
    )
    def k(y_hbm, inv_hbm, pk_hbm, res_hbm, out_hbm,
          idx_v, pk_v, y0_v, y1_v, r_v, si0, si1, sw0, sw1):
        wid = lax.axis_index("s") * nc + lax.axis_index("c")
        base = pl.multiple_of(wid * tok_w, 8)
        pltpu.sync_copy(inv_hbm.at[pl.ds(base, tok_w)], idx_v.at[0])
        pltpu.sync_copy(inv_hbm.at[pl.ds(t + base, tok_w)], idx_v.at[1])
        pltpu.sync_copy(pk_hbm.at[pl.ds(base, tok_w)], pk_v.at[0])
        pltpu.sync_copy(pk_hbm.at[pl.ds(t + base, tok_w)], pk_v.at[1])
        sems_in = (si0, si1)
        sems_w = (sw0, sw1)

        def in_copies(i, b):
            lo = pl.multiple_of(i * ch, 8)
            return (
                pltpu.make_async_copy(
                    y_hbm.at[idx_v.at[0, pl.ds(lo, ch)]], y0_v.at[b],
                    sems_in[b]),
                pltpu.make_async_copy(
                    y_hbm.at[idx_v.at[1, pl.ds(lo, ch)]], y1_v.at[b],
                    sems_in[b]),
                pltpu.make_async_copy(
                    res_hbm.at[pl.ds(base + lo, ch)], r_v.at[b], sems_in[b]),
            )

        def wb_copy(i, b):
            lo = pl.multiple_of(i * ch, 8)
            return pltpu.make_async_copy(
                r_v.at[b], out_hbm.at[pl.ds(base + lo, ch)], sems_w[b])

        for c in in_copies(0, 0):
            c.start()
        for c in in_copies(1, 1):
            c.start()

        dnums = lax.GatherDimensionNumbers(
            offset_dims=(), collapsed_slice_dims=(0,), start_index_map=(0,))

        def step(j, carry):
            for b in (0, 1):
                i = j * 2 + b
                for c in in_copies(i, b):
                    c.wait()
                w0 = pl.multiple_of((i // 2) * 16, 8)
                pk0w = pk_v[0, pl.ds(w0, 16)]
                pk1w = pk_v[1, pl.ds(w0, 16)]
                half = (i % 2) * ch

                def row_loop(r, c2):
                    lane = (jnp.zeros((16,), jnp.int32) + half + r)[:, None]
                    s0 = lax.gather(
                        pk0w, lane, dnums, (1,),
                        mode=lax.GatherScatterMode.PROMISE_IN_BOUNDS)
                    s1 = lax.gather(
                        pk1w, lane, dnums, (1,),
                        mode=lax.GatherScatterMode.PROMISE_IN_BOUNDS)

                    @plsc.parallel_loop(0, h, step=128)
                    def col_loop(c0):
                        for u in range(8):
                            slc = pl.ds(c0 + u * 16, 16)
                            r_v[b, r, slc] = (r_v[b, r, slc]
                                              + y0_v[b, r, slc] * s0
                                              + y1_v[b, r, slc] * s1)

                    return c2

                lax.fori_loop(0, ch, row_loop, 0)
                wb = wb_copy(i, b)
                wb.start()

                @pl.when(i + 2 < n_it)
                def _():
                    c0, c1, c2 = in_copies(i + 2, b)
                    c0.start()
                    c1.start()
                    wb.wait()
                    c2.start()

            return carry

        lax.fori_loop(0, n_it // 2, step, 0)
        wb_copy(n_it - 2, 0).wait()
        wb_copy(n_it - 1, 1).wait()

    return k(y, inv2, pk2, res)


def _cast_body(w_ref, o_ref):
    o_ref[...] = w_ref[...].astype(jnp.bfloat16)


def _cast_w(w):
    e, a, b = w.shape
    return pl.pallas_call(
        _cast_body,
        grid=(e,),
        in_specs=[pl.BlockSpec((1, a, b), lambda i: (i, 0, 0))],
        out_specs=pl.BlockSpec((1, a, b), lambda i: (i, 0, 0)),
        out_shape=jax.ShapeDtypeStruct((e, a, b), jnp.bfloat16),
        compiler_params=pltpu.CompilerParams(
            dimension_semantics=("arbitrary",)),
    )(w)


def _mlp_body(eid_ref, x_ref, w1_ref, w2_ref, y_ref):
    x = x_ref[...].astype(jnp.bfloat16)
    a = jnp.dot(x, w1_ref[0], preferred_element_type=jnp.float32)
    a = jax.nn.gelu(a)
    y_ref[...] = jnp.dot(a.astype(jnp.bfloat16), w2_ref[0],
                         preferred_element_type=jnp.float32)


def _grouped_mlp(xpad, eid, w1b, w2b):
    s_max, h = xpad.shape
    e, _, ff = w1b.shape
    nb = s_max // _BM
    grid_spec = pltpu.PrefetchScalarGridSpec(
        num_scalar_prefetch=1,
        grid=(nb,),
        in_specs=[
            pl.BlockSpec((_BM, h), lambda i, eid_ref: (i, 0)),
            pl.BlockSpec((1, h, ff), lambda i, eid_ref: (eid_ref[i], 0, 0)),
            pl.BlockSpec((1, ff, h), lambda i, eid_ref: (eid_ref[i], 0, 0)),
        ],
        out_specs=pl.BlockSpec((_BM, h), lambda i, eid_ref: (i, 0)),
    )
    return pl.pallas_call(
        _mlp_body,
        grid_spec=grid_spec,
        out_shape=jax.ShapeDtypeStruct((s_max, h), jnp.float32),
        compiler_params=pltpu.CompilerParams(
            dimension_semantics=("arbitrary",)),
    )(eid, xpad, w1b, w2b)


def kernel(hidden_states, mlp_residual, probs, routing_map, W1, W2):
    t, h = hidden_states.shape
    e = W1.shape[0]
    i32 = jnp.int32
    s = t * _TOPK
    nb = s // _BM + e
    s_max = nb * _BM

    rm = routing_map
    rmi = rm.astype(i32)
    counts = jnp.sum(rmi, axis=0)
    g = 128
    rmi3 = rmi.reshape(t // g, g, e).astype(jnp.float32)
    tril_incl = jnp.tril(jnp.ones((g, g), jnp.float32))
    within = jnp.einsum("jk,bke->bje", tril_incl, rmi3,
                        preferred_element_type=jnp.float32)
    bsum = rmi3.sum(axis=1)
    tril_excl = jnp.tril(jnp.ones((t // g, t // g), jnp.float32), k=-1)
    boff = jnp.einsum("jk,ke->je", tril_excl, bsum,
                      preferred_element_type=jnp.float32)
    rank = (within + boff[:, None, :]).astype(i32).reshape(t, e) - rmi
    padded = ((counts + _BM - 1) // _BM) * _BM
    ends = jnp.cumsum(padded)
    off = ends - padded
    pos = (off[None, :] + rank).astype(i32)

    inv0 = jnp.min(jnp.where(rm, pos, s_max + 1), axis=1)
    inv1 = jnp.max(jnp.where(rm, pos, -1), axis=1)
    inv2 = jnp.concatenate([inv0, inv1])

    eidx = lax.broadcasted_iota(i32, (t, e), 1)
    e0 = jnp.min(jnp.where(rm, eidx, e), axis=1)
    e1 = jnp.max(jnp.where(rm, eidx, -1), axis=1)
    pk0 = jnp.sum(jnp.where(eidx == e0[:, None], probs, 0.0), axis=1)
    pk1 = jnp.sum(jnp.where(eidx == e1[:, None], probs, 0.0), axis=1)
    pk2 = jnp.concatenate([pk0, pk1])

    blk_start = jnp.arange(nb, dtype=i32) * _BM
    eid = jnp.minimum(jnp.searchsorted(ends, blk_start, side="right"),
                      e - 1).astype(i32)

    xpad = _sc_dispatch(hidden_states, inv2, s_max)

    w1b = _cast_w(W1)
    w2b = _cast_w(W2)
    y = _grouped_mlp(xpad, eid, w1b, w2b)

    return _sc_combine(y, inv2, pk2, mlp_residual)


# speedup vs baseline: 1.1094x; 1.0043x over previous
import functools

import jax
import jax.numpy as jnp
from jax import lax
from jax.experimental import pallas as pl
from jax.experimental.pallas import tpu as pltpu
from jax.experimental.pallas import tpu_sc as plsc

_BM = 512
_NW = 32
_TOPK = 2


def _sc_dispatch(hs, inv2, s_max):
    t, h = hs.shape
    tok_w = t // _NW
    ch = 16
    n_it = tok_w // ch
    info = plsc.get_sparse_core_info()
    nc = info.num_cores
    mesh = plsc.VectorSubcoreMesh(core_axis_name="c", subcore_axis_name="s")

    @functools.partial(
        pl.kernel,
        mesh=mesh,
        out_type=jax.ShapeDtypeStruct((s_max, h), jnp.float32),
        scratch_types=[
            pltpu.VMEM((_TOPK, tok_w), jnp.int32),
            pltpu.VMEM((2, ch, h), jnp.float32),
            pltpu.SemaphoreType.DMA,
            pltpu.SemaphoreType.DMA,
            pltpu.SemaphoreType.DMA,
            pltpu.SemaphoreType.DMA,
        ],
    )
    def k(hs_hbm, inv_hbm, out_hbm, idx_v, rows_v, si0, si1, so0, so1):
        wid = lax.axis_index("s") * nc + lax.axis_index("c")
        base = pl.multiple_of(wid * tok_w, 8)
        pltpu.sync_copy(inv_hbm.at[pl.ds(base, tok_w)], idx_v.at[0])
        pltpu.sync_copy(inv_hbm.at[pl.ds(t + base, tok_w)], idx_v.at[1])
        sems_in = (si0, si1)
        sems_out = (so0, so1)

        def in_copy(i, b):
            lo = pl.multiple_of(i * ch, 8)
            return pltpu.make_async_copy(
                hs_hbm.at[pl.ds(base + lo, ch)], rows_v.at[b], sems_in[b])

        def out_copies(i, b):
            lo = pl.multiple_of(i * ch, 8)
            return (
                pltpu.make_async_copy(
                    rows_v.at[b], out_hbm.at[idx_v.at[0, pl.ds(lo, ch)]],
                    sems_out[b]),
                pltpu.make_async_copy(
                    rows_v.at[b], out_hbm.at[idx_v.at[1, pl.ds(lo, ch)]],
                    sems_out[b]),
            )

        in_copy(0, 0).start()
        in_copy(1, 1).start()

        def step(j, carry):
            for b in (0, 1):
                i = j * 2 + b
                in_copy(i, b).wait()
                c0, c1 = out_copies(i, b)
                c0.start()
                c1.start()

                @pl.when(i + 2 < n_it)
                def _():
                    c0.wait()
                    c1.wait()
                    in_copy(i + 2, b).start()

            return carry

        lax.fori_loop(0, n_it // 2, step, 0)
        for b in (0, 1):
            c0, c1 = out_copies(n_it - 2 + b, b)
            c0.wait()
            c1.wait()

    return k(hs, inv2)


def _sc_combine(y, inv2, pk2, res):
    t, h = res.shape
    tok_w = t // _NW
    ch = 8
    n_it = tok_w // ch
    info = plsc.get_sparse_core_info()
    nc = info.num_cores
    mesh = plsc.VectorSubcoreMesh(core_axis_name="c", subcore_axis_name="s")

    @functools.partial(
        pl.kernel,
        mesh=mesh,
        out_type=jax.ShapeDtypeStruct((t, h), jnp.float32),
        scratch_types=[
            pltpu.VMEM((_TOPK, tok_w), jnp.int32),
            pltpu.VMEM((_TOPK, tok_w), jnp.float32),
            pltpu.VMEM((2, ch, h), jnp.float32),
            pltpu.VMEM((2, ch, h), jnp.float32),
            pltpu.VMEM((2, ch, h), jnp.float32),
            pltpu.SemaphoreType.DMA,
            pltpu.SemaphoreType.DMA,
            pltpu.SemaphoreType.DMA,
            pltpu.SemaphoreType.DMA,
        ],
    )
    def k(y_hbm, inv_hbm, pk_hbm, res_hbm, out_hbm,
          idx_v, pk_v, y0_v, y1_v, r_v, si0, si1, sw0, sw1):
        wid = lax.axis_index("s") * nc + lax.axis_index("c")
        base = pl.multiple_of(wid * tok_w, 8)
        pltpu.sync_copy(inv_hbm.at[pl.ds(base, tok_w)], idx_v.at[0])
        pltpu.sync_copy(inv_hbm.at[pl.ds(t + base, tok_w)], idx_v.at[1])
        pltpu.sync_copy(pk_hbm.at[pl.ds(base, tok_w)], pk_v.at[0])
        pltpu.sync_copy(pk_hbm.at[pl.ds(t + base, tok_w)], pk_v.at[1])
        sems_in = (si0, si1)
        sems_w = (sw0, sw1)

        def in_copies(i, b):
            lo = pl.multiple_of(i * ch, 8)
            return (
                pltpu.make_async_copy(
                    y_hbm.at[idx_v.at[0, pl.ds(lo, ch)]], y0_v.at[b],
                    sems_in[b]),
                pltpu.make_async_copy(
                    y_hbm.at[idx_v.at[1, pl.ds(lo, ch)]], y1_v.at[b],
                    sems_in[b]),
                pltpu.make_async_copy(
                    res_hbm.at[pl.ds(base + lo, ch)], r_v.at[b], sems_in[b]),
            )

        def wb_copy(i, b):
            lo = pl.multiple_of(i * ch, 8)
            return pltpu.make_async_copy(
                r_v.at[b], out_hbm.at[pl.ds(base + lo, ch)], sems_w[b])

        for c in in_copies(0, 0):
            c.start()
        for c in in_copies(1, 1):
            c.start()

        dnums = lax.GatherDimensionNumbers(
            offset_dims=(), collapsed_slice_dims=(0,), start_index_map=(0,))

        def step(j, carry):
            for b in (0, 1):
                i = j * 2 + b
                for c in in_copies(i, b):
                    c.wait()
                w0 = pl.multiple_of((i // 2) * 16, 8)
                pk0w = pk_v[0, pl.ds(w0, 16)]
                pk1w = pk_v[1, pl.ds(w0, 16)]
                half = (i % 2) * ch

                def row_loop(r, c2):
                    lane = (jnp.zeros((16,), jnp.int32) + half + r)[:, None]
                    s0 = lax.gather(
                        pk0w, lane, dnums, (1,),
                        mode=lax.GatherScatterMode.PROMISE_IN_BOUNDS)
                    s1 = lax.gather(
                        pk1w, lane, dnums, (1,),
                        mode=lax.GatherScatterMode.PROMISE_IN_BOUNDS)

                    @plsc.parallel_loop(0, h, step=128)
                    def col_loop(c0):
                        for u in range(8):
                            slc = pl.ds(c0 + u * 16, 16)
                            r_v[b, r, slc] = (r_v[b, r, slc]
                                              + y0_v[b, r, slc] * s0
                                              + y1_v[b, r, slc] * s1)

                    return c2

                lax.fori_loop(0, ch, row_loop, 0)
                wb = wb_copy(i, b)
                wb.start()

                @pl.when(i + 2 < n_it)
                def _():
                    c0, c1, c2 = in_copies(i + 2, b)
                    c0.start()
                    c1.start()
                    wb.wait()
                    c2.start()

            return carry

        lax.fori_loop(0, n_it // 2, step, 0)
        wb_copy(n_it - 2, 0).wait()
        wb_copy(n_it - 1, 1).wait()

    return k(y, inv2, pk2, res)


def _cast_body(w_ref, o_ref):
    o_ref[...] = w_ref[...].astype(jnp.bfloat16)


def _cast_w(w):
    e, a, b = w.shape
    return pl.pallas_call(
        _cast_body,
        grid=(e,),
        in_specs=[pl.BlockSpec((1, a, b), lambda i: (i, 0, 0))],
        out_specs=pl.BlockSpec((1, a, b), lambda i: (i, 0, 0)),
        out_shape=jax.ShapeDtypeStruct((e, a, b), jnp.bfloat16),
        compiler_params=pltpu.CompilerParams(
            dimension_semantics=("arbitrary",)),
    )(w)


def _mlp_body(eid_ref, x_ref, w1_ref, w2_ref, y_ref):
    x = x_ref[...].astype(jnp.bfloat16)
    a = jnp.dot(x, w1_ref[0], preferred_element_type=jnp.float32)
    a = jax.nn.gelu(a)
    y_ref[...] = jnp.dot(a.astype(jnp.bfloat16), w2_ref[0],
                         preferred_element_type=jnp.float32)


def _grouped_mlp(xpad, eid, w1b, w2b):
    s_max, h = xpad.shape
    e, _, ff = w1b.shape
    nb = s_max // _BM
    grid_spec = pltpu.PrefetchScalarGridSpec(
        num_scalar_prefetch=1,
        grid=(nb,),
        in_specs=[
            pl.BlockSpec((_BM, h), lambda i, eid_ref: (i, 0)),
            pl.BlockSpec((1, h, ff), lambda i, eid_ref: (eid_ref[i], 0, 0)),
            pl.BlockSpec((1, ff, h), lambda i, eid_ref: (eid_ref[i], 0, 0)),
        ],
        out_specs=pl.BlockSpec((_BM, h), lambda i, eid_ref: (i, 0)),
    )
    return pl.pallas_call(
        _mlp_body,
        grid_spec=grid_spec,
        out_shape=jax.ShapeDtypeStruct((s_max, h), jnp.float32),
        compiler_params=pltpu.CompilerParams(
            dimension_semantics=("arbitrary",)),
    )(eid, xpad, w1b, w2b)


def kernel(hidden_states, mlp_residual, probs, routing_map, W1, W2):
    t, h = hidden_states.shape
    e = W1.shape[0]
    i32 = jnp.int32
    s = t * _TOPK
    nb = s // _BM + e
    s_max = nb * _BM

    rm = routing_map
    rmi = rm.astype(i32)
    counts = jnp.sum(rmi, axis=0)
    g = 128
    rmi3 = rmi.reshape(t // g, g, e).astype(jnp.float32)
    tril_incl = jnp.tril(jnp.ones((g, g), jnp.float32))
    within = jnp.einsum("jk,bke->bje", tril_incl, rmi3,
                        preferred_element_type=jnp.float32)
    bsum = rmi3.sum(axis=1)
    tril_excl = jnp.tril(jnp.ones((t // g, t // g), jnp.float32), k=-1)
    boff = jnp.einsum("jk,ke->je", tril_excl, bsum,
                      preferred_element_type=jnp.float32)
    rank = (within + boff[:, None, :]).astype(i32).reshape(t, e) - rmi
    padded = ((counts + _BM - 1) // _BM) * _BM
    ends = jnp.cumsum(padded)
    off = ends - padded
    pos = (off[None, :] + rank).astype(i32)

    inv0 = jnp.min(jnp.where(rm, pos, s_max + 1), axis=1)
    inv1 = jnp.max(jnp.where(rm, pos, -1), axis=1)
    inv2 = jnp.concatenate([inv0, inv1])

    eidx = lax.broadcasted_iota(i32, (t, e), 1)
    e0 = jnp.min(jnp.where(rm, eidx, e), axis=1)
    e1 = jnp.max(jnp.where(rm, eidx, -1), axis=1)
    pk0 = jnp.sum(jnp.where(eidx == e0[:, None], probs, 0.0), axis=1)
    pk1 = jnp.sum(jnp.where(eidx == e1[:, None], probs, 0.0), axis=1)
    pk2 = jnp.concatenate([pk0, pk1])

    blk_start = jnp.arange(nb, dtype=i32) * _BM
    eid = jnp.minimum(jnp.searchsorted(ends, blk_start, side="right"),
                      e - 1).astype(i32)

    xpad = _sc_dispatch(hidden_states, inv2, s_max)

    w1b = _cast_w(W1)
    w2b = _cast_w(W2)
    y = _grouped_mlp(xpad, eid, w1b, w2b)

    return _sc_combine(y, inv2, pk2, mlp_residual)
